# baseline (speedup 1.0000x reference)
import functools

import jax
import jax.numpy as jnp
from jax.experimental import pallas as pl
from jax.experimental.pallas import tpu as pltpu

B, L, D, D_OUT = 16, 2048, 1024, 1024


def _pool(pbuf, xbuf, idx):
    pv = pbuf[pl.ds(idx, 1), :, :][0]
    xv = xbuf[pl.ds(idx, 1), :, :][0]
    return jnp.sum(pv * xv.astype(jnp.float32), axis=0, keepdims=True)


def _body(x_ref, mask_ref, w1_ref, v_ref, wr_ref, br_ref, out_ref,
          pool_ref, xb_buf, p_buf):
    b = pl.program_id(0)
    cur = jax.lax.rem(b, 2)
    prv = jax.lax.rem(b + 1, 2)

    pooled_prev = _pool(p_buf, xb_buf, prv)
    pool_ref[pl.ds(jnp.maximum(b - 1, 0), 1), :] = pooled_prev

    x = x_ref[0]
    xb = x.astype(jnp.bfloat16)
    h = jnp.tanh(
        jax.lax.dot_general(xb, w1_ref[...], (((1,), (0,)), ((), ())),
                            preferred_element_type=jnp.float32))
    scores = jnp.sum(h * v_ref[...], axis=1, keepdims=True)
    scores = jnp.where(mask_ref[0] > 0, scores, jnp.float32(-1e9))
    m = jnp.max(scores)
    p = jnp.exp(scores - m)
    pw = p / jnp.sum(p)
    xb_buf[pl.ds(cur, 1), :, :] = xb[None]
    p_buf[pl.ds(cur, 1), :, :] = pw[None]

    @pl.when(b == B - 1)
    def _finish():
        pool_ref[pl.ds(B - 1, 1), :] = _pool(p_buf, xb_buf, cur)
        out_ref[...] = jax.lax.dot_general(
            pool_ref[...], wr_ref[...], (((1,), (0,)), ((), ())),
            preferred_element_type=jnp.float32) + br_ref[...]


@functools.partial(jax.jit, static_argnames=())
def kernel(word_hidden, word_mask, W1, v, Wr, br):
    maskf = word_mask.astype(jnp.float32).reshape(B, L, 1)
    w1_bf = W1.astype(jnp.bfloat16)
    v2 = v.reshape(1, D)
    br2 = br.reshape(1, D_OUT)
    out = pl.pallas_call(
        _body,
        grid=(B,),
        in_specs=[
            pl.BlockSpec((1, L, D), lambda b: (b, 0, 0)),
            pl.BlockSpec((1, L, 1), lambda b: (b, 0, 0)),
            pl.BlockSpec((D, D), lambda b: (0, 0)),
            pl.BlockSpec((1, D), lambda b: (0, 0)),
            pl.BlockSpec((D, D_OUT), lambda b: (0, 0)),
            pl.BlockSpec((1, D_OUT), lambda b: (0, 0)),
        ],
        out_specs=pl.BlockSpec((B, D_OUT), lambda b: (0, 0)),
        out_shape=jax.ShapeDtypeStruct((B, D_OUT), jnp.float32),
        scratch_shapes=[
            pltpu.VMEM((B, D), jnp.float32),
            pltpu.VMEM((2, L, D), jnp.bfloat16),
            pltpu.VMEM((2, L, 1), jnp.float32),
        ],
        compiler_params=pltpu.CompilerParams(
            dimension_semantics=("arbitrary",)),
    )(word_hidden, maskf, w1_bf, v2, Wr, br2)
    return out


# speedup vs baseline: 1.1517x; 1.1517x over previous
import functools

import jax
import jax.numpy as jnp
from jax.experimental import pallas as pl
from jax.experimental.pallas import tpu as pltpu

B, L, D, D_OUT = 16, 2048, 1024, 1024


def _body(x_ref, mask_ref, w1_ref, v_ref, wr_ref, br_ref, out_ref, pool_ref):
    b = pl.program_id(0)

    x = x_ref[0]
    xb = x.astype(jnp.bfloat16)
    h = jnp.tanh(
        jax.lax.dot_general(xb, w1_ref[...], (((1,), (0,)), ((), ())),
                            preferred_element_type=jnp.float32))
    scores = jnp.sum(h * v_ref[...], axis=1, keepdims=True)
    scores = jnp.where(mask_ref[0] > 0, scores, jnp.float32(-1e9))
    m = jnp.max(scores)
    p = jnp.exp(scores - m)
    pw = p / jnp.sum(p)
    pooled = jax.lax.dot_general(pw, x, (((0,), (0,)), ((), ())),
                                 preferred_element_type=jnp.float32)
    pool_ref[pl.ds(b, 1), :] = pooled

    @pl.when(b == B - 1)
    def _finish():
        out_ref[...] = jax.lax.dot_general(
            pool_ref[...], wr_ref[...], (((1,), (0,)), ((), ())),
            preferred_element_type=jnp.float32) + br_ref[...]


@functools.partial(jax.jit, static_argnames=())
def kernel(word_hidden, word_mask, W1, v, Wr, br):
    maskf = word_mask.astype(jnp.float32).reshape(B, L, 1)
    w1_bf = W1.astype(jnp.bfloat16)
    v2 = v.reshape(1, D)
    br2 = br.reshape(1, D_OUT)
    out = pl.pallas_call(
        _body,
        grid=(B,),
        in_specs=[
            pl.BlockSpec((1, L, D), lambda b: (b, 0, 0)),
            pl.BlockSpec((1, L, 1), lambda b: (b, 0, 0)),
            pl.BlockSpec((D, D), lambda b: (0, 0)),
            pl.BlockSpec((1, D), lambda b: (0, 0)),
            pl.BlockSpec((D, D_OUT), lambda b: (0, 0)),
            pl.BlockSpec((1, D_OUT), lambda b: (0, 0)),
        ],
        out_specs=pl.BlockSpec((B, D_OUT), lambda b: (0, 0)),
        out_shape=jax.ShapeDtypeStruct((B, D_OUT), jnp.float32),
        scratch_shapes=[
            pltpu.VMEM((B, D), jnp.float32),
        ],
        compiler_params=pltpu.CompilerParams(
            dimension_semantics=("arbitrary",)),
    )(word_hidden, maskf, w1_bf, v2, Wr, br2)
    return out
